# Initial kernel scaffold; baseline (speedup 1.0000x reference)
import jax
import jax.numpy as jnp
from jax.experimental import pallas as pl


def kernel(t, context, x, pos, eigvecs, edge_attr, params, edge_index, batch_ids):
    raise NotImplementedError("write your pallas kernel here")


# speedup vs baseline: 1.1864x; 1.1864x over previous
import math
import functools

import jax
import jax.numpy as jnp
from jax.experimental import pallas as pl
from jax.experimental.pallas import tpu as pltpu

N_NODES_C = 10000
N_EDGES_C = 160000
N_GRAPHS_C = 16
TIME_DIM_C = 16

EDGE_BLOCK = 3200


def _silu(x):
    return x * jax.nn.sigmoid(x)


def _timestep_embedding(timesteps, dim, max_period=10000):
    half = dim // 2
    freqs = jnp.exp(-math.log(max_period) * jnp.arange(0, half, dtype=jnp.float32) / half)
    args = timesteps[:, None].astype(jnp.float32) * freqs[None]
    return jnp.concatenate([jnp.cos(args), jnp.sin(args)], axis=-1)


def _edge_block_kernel(hrow_ref, hcol_ref, cd_ref, eattr_ref,
                       w2_ref, b2_ref, wc0_ref, bc0_ref, wc1_ref, m4_ref,
                       m_ref, trans_ref):
    cd = cd_ref[...]
    radial = jnp.sum(cd * cd, axis=1, keepdims=True)
    ea = eattr_ref[...]
    pre1 = (hrow_ref[...] + hcol_ref[...]
            + jnp.concatenate([ea, radial], axis=1) @ m4_ref[...])
    t1 = _silu(pre1)
    m = _silu(t1 @ w2_ref[...] + b2_ref[...])
    q = _silu(m @ wc0_ref[...] + bc0_ref[...])
    s = jnp.sum(q * wc1_ref[...], axis=1, keepdims=True)
    m_ref[...] = m
    trans_ref[...] = cd * s


def _run_edge_block(hrow, hcol, cd, eattr, w2, b2, wc0, bc0, wc1, m4):
    n_edges = hrow.shape[0]
    grid = n_edges // EDGE_BLOCK
    eb = EDGE_BLOCK
    bs_e = lambda w: pl.BlockSpec((eb, w), lambda i: (i, 0))
    bs_c = lambda a, b: pl.BlockSpec((a, b), lambda i: (0, 0))
    m, trans = pl.pallas_call(
        _edge_block_kernel,
        grid=(grid,),
        in_specs=[bs_e(64), bs_e(64), bs_e(3), bs_e(4),
                  bs_c(64, 64), bs_c(1, 64), bs_c(64, 64), bs_c(1, 64),
                  bs_c(1, 64), bs_c(5, 64)],
        out_specs=[bs_e(64), bs_e(3)],
        out_shape=[jax.ShapeDtypeStruct((n_edges, 64), jnp.float32),
                   jax.ShapeDtypeStruct((n_edges, 3), jnp.float32)],
    )(hrow, hcol, cd, eattr, w2, b2, wc0, bc0, wc1, m4)
    return m, trans


def kernel(t, context, x, pos, eigvecs, edge_attr, params, edge_index, batch_ids):
    with jax.default_matmul_precision("float32"):
        return _forward_impl(t, context, x, pos, eigvecs, edge_attr, params,
                             edge_index, batch_ids)


def _forward_impl(t, context, x, pos, eigvecs, edge_attr, params, edge_index, batch_ids):
    f32 = jnp.float32
    pe = jnp.where(jnp.isnan(eigvecs), 0.0, eigvecs) @ params["pe_enc"]["W"] + params["pe_enc"]["b"]
    tg = _timestep_embedding(t, TIME_DIM_C)
    time_emb = tg[batch_ids]
    ctx = (context @ params["context_emb"]["W"] + params["context_emb"]["b"])[batch_ids]
    h_node = x @ params["node_emb"]["W"] + params["node_emb"]["b"]
    h = jnp.concatenate([h_node, pe, time_emb, ctx], axis=1)

    row = edge_index[0]
    col = edge_index[1]
    n = h.shape[0]

    ttab = tg[batch_ids[:N_GRAPHS_C]]

    we = params["edge_emb"]["W"]
    be = params["edge_emb"]["b"]

    ones = jnp.ones((row.shape[0],), f32)
    cnt = jax.ops.segment_sum(ones, row, num_segments=n)[:, None]
    inv_cnt = 1.0 / jnp.maximum(cnt, 1.0)

    conv = params["convs"][0]
    h = h @ conv["emb_in"]["W"] + conv["emb_in"]["b"]
    p = pos

    for gcl in conv["gcls"]:
        w1 = gcl["edge_mlp"][0]["W"]
        b1 = gcl["edge_mlp"][0]["b"]
        w1a, w1b = w1[0:64], w1[64:128]
        wr = w1[128:129]
        w1e = w1[129:177]
        w1t = w1[177:193]
        m4 = jnp.concatenate([we @ w1e, wr], axis=0)
        tvec = ttab @ w1t + (be @ w1e + b1)[None, :]
        hA = h @ w1a + tvec[batch_ids]
        hB = h @ w1b

        hrow = hA[row]
        hcol = hB[col]
        cd = p[row] - p[col]

        w2, b2 = gcl["edge_mlp"][1]["W"], gcl["edge_mlp"][1]["b"]
        wc0, bc0 = gcl["coord_mlp"][0]["W"], gcl["coord_mlp"][0]["b"]
        wc1 = gcl["coord_mlp"][1]["W"].T
        m, trans = _run_edge_block(hrow, hcol, cd, edge_attr, w2,
                                   b2[None, :], wc0, bc0[None, :], wc1, m4)

        p = p + jax.ops.segment_sum(trans, row, num_segments=n) * inv_cnt
        agg = jax.ops.segment_sum(m, row, num_segments=n)

        wn0, bn0 = gcl["node_mlp"][0]["W"], gcl["node_mlp"][0]["b"]
        wn1, bn1 = gcl["node_mlp"][1]["W"], gcl["node_mlp"][1]["b"]
        hid = _silu(h @ wn0[:64] + agg @ wn0[64:] + bn0)
        h = h + (hid @ wn1 + bn1)

    h = h @ conv["emb_out"]["W"] + conv["emb_out"]["b"]

    hg = jax.ops.segment_sum(h, batch_ids, num_segments=N_GRAPHS_C)
    mlp = params["mlp"]
    out = jax.nn.relu(hg @ mlp[0]["W"] + mlp[0]["b"])
    out = jax.nn.relu(out @ mlp[1]["W"] + mlp[1]["b"])
    out = out @ mlp[2]["W"] + mlp[2]["b"]
    return out
